# Initial kernel scaffold; baseline (speedup 1.0000x reference)
import jax
import jax.numpy as jnp
from jax.experimental import pallas as pl


def kernel(target, context, embd):
    raise NotImplementedError("write your pallas kernel here")


# speedup vs baseline: 15.4026x; 15.4026x over previous
import functools

import jax
import jax.numpy as jnp
from jax import lax
from jax.experimental import pallas as pl
from jax.experimental.pallas import tpu as pltpu
from jax.experimental.pallas import tpu_sc as plsc

EMB = 128
VP = 1024
NC, NS, L = 2, 16, 16
NW = NC * NS
IDX_W = 128
GCHUNK = 8


def _tc_body(emb_ref, tgt_ref, ctx_ref, gram_ref, idx_ref):
    a = emb_ref[...]
    gram_ref[...] = lax.dot_general(
        a, a, (((1,), (1,)), ((), ())), preferred_element_type=jnp.float32)
    idx_ref[...] = tgt_ref[...] * VP + ctx_ref[...]


def _tc_call(emb_padded, target2d, context):
    B, C = context.shape
    return pl.pallas_call(
        _tc_body,
        out_shape=(jax.ShapeDtypeStruct((VP, VP), jnp.float32),
                   jax.ShapeDtypeStruct((B, C), jnp.int32)),
    )(emb_padded, target2d, context)


def _make_sc_gather(B, C):
    assert B * C % NW == 0
    PAIRS = B * C // NW
    assert PAIRS % IDX_W == 0
    NROW = PAIRS // IDX_W
    assert NROW % GCHUNK == 0

    mesh = plsc.VectorSubcoreMesh(
        core_axis_name="c", subcore_axis_name="s",
        num_cores=NC, num_subcores=NS)

    @functools.partial(
        pl.kernel,
        out_type=jax.ShapeDtypeStruct((B * C,), jnp.float32),
        mesh=mesh,
        scratch_types=[
            pltpu.VMEM((PAIRS,), jnp.int32),
            pltpu.VMEM((PAIRS,), jnp.float32),
            pltpu.SemaphoreType.DMA,
        ],
    )
    def sc_gather(gram_hbm, idx_hbm, out_hbm, idx_v, rows_v, sem):
        wid = lax.axis_index("s") * NC + lax.axis_index("c")
        base = wid * PAIRS
        pltpu.sync_copy(idx_hbm.at[pl.ds(base, PAIRS)], idx_v)

        def gather_chunk(ck, _):
            j0 = ck * GCHUNK
            for u in range(GCHUNK):
                pltpu.async_copy(
                    gram_hbm.at[idx_v.at[pl.ds((j0 + u) * IDX_W, IDX_W)]],
                    rows_v.at[pl.ds((j0 + u) * IDX_W, IDX_W)], sem)
            for u in range(GCHUNK):
                pltpu.make_async_copy(
                    gram_hbm.at[idx_v.at[pl.ds((j0 + u) * IDX_W, IDX_W)]],
                    rows_v.at[pl.ds((j0 + u) * IDX_W, IDX_W)], sem).wait()
            return 0

        lax.fori_loop(0, NROW // GCHUNK, gather_chunk, 0)
        pltpu.sync_copy(rows_v, out_hbm.at[pl.ds(base, PAIRS)])

    return sc_gather


def kernel(target, context, embd):
    B, = target.shape
    C = context.shape[1]
    ep = jnp.zeros((VP, EMB), embd.dtype).at[: embd.shape[0]].set(embd)
    g, idx = _tc_call(ep, target.reshape(B, 1), context)
    out = _make_sc_gather(B, C)(g.reshape(VP * VP), idx.reshape(B * C))
    return out.reshape(B, C)


# speedup vs baseline: 27.1772x; 1.7645x over previous
import functools

import jax
import jax.numpy as jnp
from jax import lax
from jax.experimental import pallas as pl
from jax.experimental.pallas import tpu as pltpu
from jax.experimental.pallas import tpu_sc as plsc

EMB = 128
VP = 1024
NC, NS, L = 2, 16, 16
NW = NC * NS
IDX_W = 128
GCHUNK = 8
KBLK = 8


def _tc_body(a_full, a_blk, tgt_ref, ctxT_ref, gram_ref, idx_ref):
    k = pl.program_id(0)
    g = lax.dot_general(
        a_full[...], a_blk[...], (((1,), (1,)), ((), ())),
        preferred_element_type=jnp.float32)
    gram_ref[...] = g.reshape(1, VP, VP // KBLK)

    @pl.when(k == 0)
    def _():
        ctx = ctxT_ref[...]
        idx_ref[pl.ds(0, ctxT_ref.shape[0]), :] = (
            (ctx >> 7) * (VP * (VP // KBLK))
            + tgt_ref[...] * (VP // KBLK) + (ctx & (VP // KBLK - 1)))


def _tc_call(emb_padded, target_row, contextT):
    C, B = contextT.shape
    CP = (C + 7) // 8 * 8
    return pl.pallas_call(
        _tc_body,
        grid=(KBLK,),
        in_specs=[
            pl.BlockSpec((VP, EMB), lambda k: (0, 0)),
            pl.BlockSpec((VP // KBLK, EMB), lambda k: (k, 0)),
            pl.BlockSpec((1, B), lambda k: (0, 0)),
            pl.BlockSpec((C, B), lambda k: (0, 0)),
        ],
        out_specs=[
            pl.BlockSpec((1, VP, VP // KBLK), lambda k: (k, 0, 0)),
            pl.BlockSpec((CP, B), lambda k: (0, 0)),
        ],
        out_shape=(jax.ShapeDtypeStruct((KBLK, VP, VP // KBLK), jnp.float32),
                   jax.ShapeDtypeStruct((CP, B), jnp.int32)),
    )(emb_padded, emb_padded, target_row, contextT)


def _make_sc_gather(B, C):
    assert B * C % NW == 0
    PAIRS = B * C // NW
    assert PAIRS % IDX_W == 0
    NROW = PAIRS // IDX_W
    assert NROW % GCHUNK == 0

    mesh = plsc.VectorSubcoreMesh(
        core_axis_name="c", subcore_axis_name="s",
        num_cores=NC, num_subcores=NS)

    @functools.partial(
        pl.kernel,
        out_type=jax.ShapeDtypeStruct((B * C,), jnp.float32),
        mesh=mesh,
        scratch_types=[
            pltpu.VMEM((PAIRS,), jnp.int32),
            pltpu.VMEM((PAIRS,), jnp.float32),
            pltpu.SemaphoreType.DMA,
        ],
    )
    def sc_gather(gram_hbm, idx_hbm, out_hbm, idx_v, rows_v, sem):
        wid = lax.axis_index("s") * NC + lax.axis_index("c")
        base = wid * PAIRS
        pltpu.sync_copy(idx_hbm.at[pl.ds(base, PAIRS)], idx_v)

        def gather_chunk(ck, _):
            j0 = ck * GCHUNK
            for u in range(GCHUNK):
                pltpu.async_copy(
                    gram_hbm.at[idx_v.at[pl.ds((j0 + u) * IDX_W, IDX_W)]],
                    rows_v.at[pl.ds((j0 + u) * IDX_W, IDX_W)], sem)
            for u in range(GCHUNK):
                pltpu.make_async_copy(
                    gram_hbm.at[idx_v.at[pl.ds((j0 + u) * IDX_W, IDX_W)]],
                    rows_v.at[pl.ds((j0 + u) * IDX_W, IDX_W)], sem).wait()
            return 0

        lax.fori_loop(0, NROW // GCHUNK, gather_chunk, 0)
        pltpu.sync_copy(rows_v, out_hbm.at[pl.ds(base, PAIRS)])

    return sc_gather


def kernel(target, context, embd):
    B, = target.shape
    C = context.shape[1]
    CP = (C + 7) // 8 * 8
    ep = jnp.zeros((VP, EMB), embd.dtype).at[: embd.shape[0]].set(embd)
    g, idxT = _tc_call(ep, target.reshape(1, B), context.T)
    out = _make_sc_gather(B, C)(g.reshape(VP * VP), idxT.reshape(CP * B))
    return out.reshape(C, B).T


# speedup vs baseline: 31.1986x; 1.1480x over previous
import functools

import jax
import jax.numpy as jnp
from jax import lax
from jax.experimental import pallas as pl
from jax.experimental.pallas import tpu as pltpu
from jax.experimental.pallas import tpu_sc as plsc

EMB = 128
VP = 1024
NC, NS, L = 2, 16, 16
NW = NC * NS
IDX_W = 128
GCHUNK = 8
KBLK = 8


def _tc_body(a_full, a_blk, tgt_ref, ctxT_ref, gram_ref, idx_ref):
    k = pl.program_id(0)
    g = lax.dot_general(
        a_full[...], a_blk[...], (((1,), (1,)), ((), ())),
        preferred_element_type=jnp.float32)
    gram_ref[...] = g.reshape(1, VP, VP // KBLK)

    @pl.when(k == 0)
    def _():
        ctx = ctxT_ref[...]
        idx_ref[pl.ds(0, ctxT_ref.shape[0]), :] = (
            (ctx >> 7) * (VP * (VP // KBLK))
            + tgt_ref[...] * (VP // KBLK) + (ctx & (VP // KBLK - 1)))


def _tc_call(emb_padded, target_row, contextT):
    C, B = contextT.shape
    CP = (C + 7) // 8 * 8
    return pl.pallas_call(
        _tc_body,
        grid=(KBLK,),
        in_specs=[
            pl.BlockSpec((VP, EMB), lambda k: (0, 0)),
            pl.BlockSpec((VP // KBLK, EMB), lambda k: (k, 0)),
            pl.BlockSpec((1, B), lambda k: (0, 0)),
            pl.BlockSpec((C, B), lambda k: (0, 0)),
        ],
        out_specs=[
            pl.BlockSpec((1, VP, VP // KBLK), lambda k: (k, 0, 0)),
            pl.BlockSpec((CP, B), lambda k: (0, 0)),
        ],
        out_shape=(jax.ShapeDtypeStruct((KBLK, VP, VP // KBLK), jnp.float32),
                   jax.ShapeDtypeStruct((CP, B), jnp.int32)),
    )(emb_padded, emb_padded, target_row, contextT)


def _make_sc_gather(B, C):
    assert B * C % NW == 0
    PAIRS = B * C // NW
    assert PAIRS % IDX_W == 0
    NROW = PAIRS // IDX_W
    assert NROW % GCHUNK == 0

    mesh = plsc.VectorSubcoreMesh(
        core_axis_name="c", subcore_axis_name="s",
        num_cores=NC, num_subcores=NS)

    @functools.partial(
        pl.kernel,
        out_type=jax.ShapeDtypeStruct((B * C,), jnp.float32),
        mesh=mesh,
        scratch_types=[
            pltpu.VMEM((PAIRS,), jnp.int32),
            pltpu.VMEM((PAIRS,), jnp.float32),
            pltpu.SemaphoreType.DMA,
        ],
    )
    def sc_gather(gram_hbm, idx_hbm, out_hbm, idx_v, rows_v, sem):
        wid = lax.axis_index("s") * NC + lax.axis_index("c")
        base = wid * PAIRS
        pltpu.sync_copy(idx_hbm.at[pl.ds(base, PAIRS)], idx_v)

        pltpu.async_copy(gram_hbm.at[idx_v], rows_v, sem)
        pltpu.make_async_copy(gram_hbm.at[idx_v], rows_v, sem).wait()
        pltpu.sync_copy(rows_v, out_hbm.at[pl.ds(base, PAIRS)])

    return sc_gather


def kernel(target, context, embd):
    B, = target.shape
    C = context.shape[1]
    CP = (C + 7) // 8 * 8
    ep = jnp.zeros((VP, EMB), embd.dtype).at[: embd.shape[0]].set(embd)
    g, idxT = _tc_call(ep, target.reshape(1, B), context.T)
    out = _make_sc_gather(B, C)(g.reshape(VP * VP), idxT.reshape(CP * B))
    return out.reshape(C, B).T
